# Initial kernel scaffold; baseline (speedup 1.0000x reference)
import jax
import jax.numpy as jnp
from jax.experimental import pallas as pl


def kernel(coords, feats, W1a, b1a, W1b, b1b, W2, b2, Wh1, bh1, Wh2, bh2):
    raise NotImplementedError("write your pallas kernel here")


# speedup vs baseline: 1.8942x; 1.8942x over previous
import functools

import jax
import jax.numpy as jnp
from jax import lax
from jax.experimental import pallas as pl
from jax.experimental.pallas import tpu as pltpu
from jax.experimental.pallas import tpu_sc as plsc

N = 32768
B = 16
F2 = 64
NC = 2
NS = 16
NW = NC * NS
CHUNK = N // NW


ENC_BLK = 4096


def _encoder_body(feats_ref, w1a_ref, b1a_ref, w1b_ref, b1b_ref, w2_ref,
                  b2_ref, out_ref):
    x = feats_ref[...]
    h = jnp.dot(x, w1a_ref[...], preferred_element_type=jnp.float32)
    h = jnp.maximum(h + b1a_ref[...], 0.0)
    h = jnp.dot(h, w1b_ref[...], preferred_element_type=jnp.float32)
    h = jnp.maximum(h + b1b_ref[...], 0.0)
    h = jnp.dot(h, w2_ref[...], preferred_element_type=jnp.float32)
    h = jnp.maximum(h + b2_ref[...], 0.0)
    out_ref[...] = h


def _encoder(feats, W1a, b1a, W1b, b1b, W2, b2):
    grid = (N // ENC_BLK,)
    full = lambda shape: pl.BlockSpec(shape, lambda i: (0, 0))
    return pl.pallas_call(
        _encoder_body,
        grid=grid,
        in_specs=[
            pl.BlockSpec((ENC_BLK, 8), lambda i: (i, 0)),
            full((8, 16)), full((1, 16)),
            full((16, 32)), full((1, 32)),
            full((32, 64)), full((1, 64)),
        ],
        out_specs=pl.BlockSpec((ENC_BLK, F2), lambda i: (i, 0)),
        out_shape=jax.ShapeDtypeStruct((N, F2), jnp.float32),
    )(feats, W1a, b1a.reshape(1, 16), W1b, b1b.reshape(1, 32),
      W2, b2.reshape(1, 64))


def _pool_body(bi_hbm, f2_hbm, out_hbm, idx_v, rows_v, sums_v):
    wid = lax.axis_index("s") * NC + lax.axis_index("c")
    base = wid * CHUNK
    pltpu.sync_copy(bi_hbm.at[pl.ds(base, CHUNK)], idx_v)
    pltpu.sync_copy(f2_hbm.at[pl.ds(base * F2, CHUNK * F2)], rows_v)

    zero = jnp.zeros((16,), jnp.float32)
    for i in range(B * F2 // 16):
        sums_v[pl.ds(i * 16, 16)] = zero

    def body(g, carry):
        bv = idx_v[pl.ds(g * 16, 16)]
        for l in range(16):
            b = bv[l]
            i = g * 16 + l
            for j in range(F2 // 16):
                sums_v[pl.ds(b * F2 + j * 16, 16)] += (
                    rows_v[pl.ds(i * F2 + j * 16, 16)])
        return carry

    lax.fori_loop(0, CHUNK // 16, body, 0)
    pltpu.sync_copy(sums_v, out_hbm.at[pl.ds(wid * B * F2, B * F2)])


def _pool(batch_idx, f2):
    mesh = plsc.VectorSubcoreMesh(core_axis_name="c", subcore_axis_name="s")
    f = functools.partial(
        pl.kernel,
        out_type=jax.ShapeDtypeStruct((NW * B * F2,), jnp.float32),
        mesh=mesh,
        scratch_types=[
            pltpu.VMEM((CHUNK,), jnp.int32),
            pltpu.VMEM((CHUNK * F2,), jnp.float32),
            pltpu.VMEM((B * F2,), jnp.float32),
        ],
    )(_pool_body)
    return f(batch_idx, f2.reshape(N * F2))


def _head_body(part_ref, bi_ref, wh1_ref, bh1_ref, wh2_ref, bh2_ref, out_ref):
    sums = jnp.sum(part_ref[...].reshape(NW, B, F2), axis=0)
    bi = bi_ref[...]
    counts = [jnp.sum(jnp.where(bi == b, 1.0, 0.0)) for b in range(B)]
    counts = jnp.stack(counts).reshape(B, 1)
    z = sums / jnp.maximum(counts, 1.0)
    h = jnp.dot(z, wh1_ref[...], preferred_element_type=jnp.float32)
    h = jnp.maximum(h + bh1_ref[...], 0.0)
    out_ref[...] = (jnp.dot(h, wh2_ref[...], preferred_element_type=jnp.float32)
                    + bh2_ref[...])


def _head(partials, batch_idx, Wh1, bh1, Wh2, bh2):
    return pl.pallas_call(
        _head_body,
        out_shape=jax.ShapeDtypeStruct((B, 2), jnp.float32),
    )(partials.reshape(NW * B, F2), batch_idx.reshape(256, 128),
      Wh1, bh1.reshape(1, 64), Wh2, bh2.reshape(1, 2))


def kernel(coords, feats, W1a, b1a, W1b, b1b, W2, b2, Wh1, bh1, Wh2, bh2):
    batch_idx = coords[:, 0]
    f2 = _encoder(feats, W1a, b1a, W1b, b1b, W2, b2)
    partials = _pool(batch_idx, f2).reshape(NW, B, F2)
    return _head(partials, batch_idx, Wh1, bh1, Wh2, bh2)


# speedup vs baseline: 2.5175x; 1.3291x over previous
import functools

import jax
import jax.numpy as jnp
from jax import lax
from jax.experimental import pallas as pl
from jax.experimental.pallas import tpu as pltpu
from jax.experimental.pallas import tpu_sc as plsc

N = 32768
B = 16
F2 = 64
NC = 2
NS = 16
NW = NC * NS

ENC_BLK = 4096
GRID = N // ENC_BLK
ROWS = N // 2
SUB_ROWS = 512
SLOT = 128


def _encoder_body(coords_ref, feats_ref, w1a_ref, b1a_ref, w1b_ref, b1b_ref,
                  w2_ref, b2_ref, out_ref, bi_ref):
    x = feats_ref[...]
    h = jnp.dot(w1a_ref[...], x, preferred_element_type=jnp.float32)
    h = jnp.maximum(h + b1a_ref[...], 0.0)
    h = jnp.dot(w1b_ref[...], h, preferred_element_type=jnp.float32)
    h = jnp.maximum(h + b1b_ref[...], 0.0)
    h = jnp.dot(w2_ref[...], h, preferred_element_type=jnp.float32)
    h = jnp.maximum(h + b2_ref[...], 0.0)
    t = h.T
    out_ref[...] = jnp.concatenate(
        [t[:ENC_BLK // 2], t[ENC_BLK // 2:]], axis=1)
    bi_ref[...] = coords_ref[...][0, :].reshape(ENC_BLK // 128, 128)


def _encoder(coords, feats, W1a, b1a, W1b, b1b, W2, b2):
    full = lambda shape: pl.BlockSpec(shape, lambda i: (0, 0))
    return pl.pallas_call(
        _encoder_body,
        grid=(GRID,),
        in_specs=[
            pl.BlockSpec((3, ENC_BLK), lambda i: (0, i)),
            pl.BlockSpec((8, ENC_BLK), lambda i: (0, i)),
            full((16, 8)), full((16, 1)),
            full((32, 16)), full((32, 1)),
            full((64, 32)), full((64, 1)),
        ],
        out_specs=(
            pl.BlockSpec((ENC_BLK // 2, 128), lambda i: (i, 0)),
            pl.BlockSpec((ENC_BLK // 128, 128), lambda i: (i, 0)),
        ),
        out_shape=(
            jax.ShapeDtypeStruct((ROWS, 128), jnp.float32),
            jax.ShapeDtypeStruct((N // 128, 128), jnp.int32),
        ),
    )(coords.T, feats.T, W1a.T, b1a.reshape(16, 1), W1b.T, b1b.reshape(32, 1),
      W2.T, b2.reshape(64, 1))


def _pool_body(bi_hbm, f2_hbm, out_hbm, idx_v, rows_v, sums_v):
    wid = lax.axis_index("s") * NC + lax.axis_index("c")
    blk = wid // 4
    r0 = (wid % 4) * SUB_ROWS
    row_base = blk * (ENC_BLK // 2) + r0
    pa = blk * ENC_BLK + r0
    pb = blk * ENC_BLK + ENC_BLK // 2 + r0
    pltpu.sync_copy(bi_hbm.at[pl.ds(pa, SUB_ROWS)], idx_v.at[pl.ds(0, SUB_ROWS)])
    pltpu.sync_copy(bi_hbm.at[pl.ds(pb, SUB_ROWS)],
                    idx_v.at[pl.ds(SUB_ROWS, SUB_ROWS)])
    pltpu.sync_copy(f2_hbm.at[pl.ds(row_base * 128, SUB_ROWS * 128)], rows_v)

    zero = jnp.zeros((16,), jnp.float32)
    for i in range(B * SLOT // 16):
        sums_v[pl.ds(i * 16, 16)] = zero

    def body(g, carry):
        bva = idx_v[pl.ds(g * 16, 16)]
        bvb = idx_v[pl.ds(SUB_ROWS + g * 16, 16)]
        for l in range(16):
            r = g * 16 + l
            ba = bva[l]
            bb = bvb[l]
            for j in range(4):
                sums_v[pl.ds(ba * SLOT + j * 16, 16)] += (
                    rows_v[pl.ds(r * 128 + j * 16, 16)])
            for j in range(4):
                sums_v[pl.ds(bb * SLOT + 64 + j * 16, 16)] += (
                    rows_v[pl.ds(r * 128 + 64 + j * 16, 16)])
        return carry

    lax.fori_loop(0, SUB_ROWS // 16, body, 0)
    pltpu.sync_copy(sums_v, out_hbm.at[pl.ds(wid * B * SLOT, B * SLOT)])


def _pool(batch_idx_flat, f2_flat):
    mesh = plsc.VectorSubcoreMesh(core_axis_name="c", subcore_axis_name="s")
    f = functools.partial(
        pl.kernel,
        out_type=jax.ShapeDtypeStruct((NW * B * SLOT,), jnp.float32),
        mesh=mesh,
        scratch_types=[
            pltpu.VMEM((2 * SUB_ROWS,), jnp.int32),
            pltpu.VMEM((SUB_ROWS * 128,), jnp.float32),
            pltpu.VMEM((B * SLOT,), jnp.float32),
        ],
    )(_pool_body)
    return f(batch_idx_flat, f2_flat)


def _head_body(part_ref, bi_ref, wh1_ref, bh1_ref, wh2_ref, bh2_ref, out_ref):
    x = part_ref[...]
    rows = lax.broadcasted_iota(jnp.int32, (B, NW * B), 1)
    sel = (rows % B == lax.broadcasted_iota(jnp.int32, (B, NW * B), 0))
    s = jnp.dot(sel.astype(jnp.float32), x,
                preferred_element_type=jnp.float32)
    sums = s[:, :F2] + s[:, F2:]
    bi = bi_ref[...]
    counts = [jnp.sum(jnp.where(bi == b, 1.0, 0.0)) for b in range(B)]
    counts = jnp.stack(counts).reshape(B, 1)
    z = sums / jnp.maximum(counts, 1.0)
    h = jnp.dot(z, wh1_ref[...], preferred_element_type=jnp.float32)
    h = jnp.maximum(h + bh1_ref[...], 0.0)
    out_ref[...] = (jnp.dot(h, wh2_ref[...], preferred_element_type=jnp.float32)
                    + bh2_ref[...])


def _head(partials, bi_arr, Wh1, bh1, Wh2, bh2):
    return pl.pallas_call(
        _head_body,
        out_shape=jax.ShapeDtypeStruct((B, 2), jnp.float32),
    )(partials.reshape(NW * B, SLOT), bi_arr,
      Wh1, bh1.reshape(1, 64), Wh2, bh2.reshape(1, 2))


def kernel(coords, feats, W1a, b1a, W1b, b1b, W2, b2, Wh1, bh1, Wh2, bh2):
    f2p, bip = _encoder(coords, feats, W1a, b1a, W1b, b1b, W2, b2)
    partials = _pool(bip.reshape(N), f2p.reshape(ROWS * 128))
    return _head(partials, bip, Wh1, bh1, Wh2, bh2)


# speedup vs baseline: 4.0570x; 1.6115x over previous
import functools

import jax
import jax.numpy as jnp
from jax import lax
from jax.experimental import pallas as pl
from jax.experimental.pallas import tpu as pltpu
from jax.experimental.pallas import tpu_sc as plsc

N = 32768
B = 16
F2 = 64
NC = 2
NS = 16
NW = NC * NS

ENC_BLK = 4096
GRID = N // ENC_BLK
ROWS = N // 2
CHUNK = 1024


def _encoder_body(coords_ref, feats_ref, w1a_ref, b1a_ref, w1b_ref, b1b_ref,
                  w2_ref, b2_ref, out_ref, bi_ref):
    x = feats_ref[...]
    cn = (((0,), (0,)), ((), ()))
    h = lax.dot_general(w1a_ref[...], x, cn, preferred_element_type=jnp.float32)
    h = jnp.maximum(h + jnp.transpose(b1a_ref[...]), 0.0)
    h = lax.dot_general(w1b_ref[...], h, cn, preferred_element_type=jnp.float32)
    h = jnp.maximum(h + jnp.transpose(b1b_ref[...]), 0.0)
    h = lax.dot_general(w2_ref[...], h, cn, preferred_element_type=jnp.float32)
    h = jnp.maximum(h + jnp.transpose(b2_ref[...]), 0.0)
    t = h.T
    out_ref[...] = jnp.concatenate(
        [t[:ENC_BLK // 2], t[ENC_BLK // 2:]], axis=1)
    bi_ref[...] = coords_ref[...][0, :].reshape(ENC_BLK // 128, 128)


def _encoder(coords, feats, W1a, b1a, W1b, b1b, W2, b2):
    full = lambda shape: pl.BlockSpec(shape, lambda i: (0, 0))
    return pl.pallas_call(
        _encoder_body,
        grid=(GRID,),
        in_specs=[
            pl.BlockSpec((3, ENC_BLK), lambda i: (0, i)),
            pl.BlockSpec((8, ENC_BLK), lambda i: (0, i)),
            full((8, 16)), full((1, 16)),
            full((16, 32)), full((1, 32)),
            full((32, 64)), full((1, 64)),
        ],
        out_specs=(
            pl.BlockSpec((ENC_BLK // 2, 128), lambda i: (i, 0)),
            pl.BlockSpec((ENC_BLK // 128, 128), lambda i: (i, 0)),
        ),
        out_shape=(
            jax.ShapeDtypeStruct((ROWS, 128), jnp.float32),
            jax.ShapeDtypeStruct((N // 128, 128), jnp.int32),
        ),
    )(coords.T, feats.T, W1a, b1a.reshape(1, 16), W1b, b1b.reshape(1, 32),
      W2, b2.reshape(1, 64))


def _pool_body(bi_hbm, f2_hbm, out_hbm, idx_v, rows_v, zer_v, shared):
    c = lax.axis_index("c")
    s = lax.axis_index("s")
    wid = s * NC + c
    half = wid // 16
    t = wid % 16
    row0 = t * CHUNK
    p0 = (t // 2) * ENC_BLK + half * (ENC_BLK // 2) + (t % 2) * CHUNK

    pltpu.sync_copy(bi_hbm.at[pl.ds(p0, CHUNK)], idx_v)
    pltpu.sync_copy(f2_hbm.at[pl.ds(row0, CHUNK), pl.ds(half * F2, F2)],
                    rows_v)

    zero = jnp.zeros((16,), jnp.float32)
    for i in range(B):
        for j in range(F2 // 16):
            zer_v[i, pl.ds(j * 16, 16)] = zero
    pltpu.sync_copy(zer_v, shared.at[pl.ds(s * B, B), :])

    base = s * B
    for g in range(CHUNK // 16):
        idx_v[pl.ds(g * 16, 16)] = idx_v[pl.ds(g * 16, 16)] + base
    pltpu.sync_copy(rows_v, shared.at[idx_v], add=True)

    pltpu.sync_copy(shared.at[pl.ds(s * B, B), :],
                    out_hbm.at[pl.ds(wid * B, B), :])


def _pool(batch_idx_flat, f2_rows):
    mesh = plsc.VectorSubcoreMesh(core_axis_name="c", subcore_axis_name="s")
    f = functools.partial(
        pl.kernel,
        out_type=jax.ShapeDtypeStruct((NW * B, F2), jnp.float32),
        mesh=mesh,
        scratch_types=[
            pltpu.VMEM((CHUNK,), jnp.int32),
            pltpu.VMEM((CHUNK, F2), jnp.float32),
            pltpu.VMEM((B, F2), jnp.float32),
            pltpu.VMEM_SHARED((NS * B, F2), jnp.float32),
        ],
        compiler_params=pltpu.CompilerParams(use_tc_tiling_on_sc=False),
    )(_pool_body)
    return f(batch_idx_flat, f2_rows)


def _head_body(part_ref, bi_ref, wh1_ref, bh1_ref, wh2t_ref, bh2_ref, out_ref):
    x = part_ref[...]
    nr = NW * B // 2
    r = lax.broadcasted_iota(jnp.int32, (B, nr), 1)
    bcol = lax.broadcasted_iota(jnp.int32, (B, nr), 0)
    sel_e = ((2 * r) % B == bcol).astype(jnp.float32)
    sel_o = ((2 * r + 1) % B == bcol).astype(jnp.float32)
    se = jnp.dot(sel_e, x, preferred_element_type=jnp.float32)
    so = jnp.dot(sel_o, x, preferred_element_type=jnp.float32)
    sums = se[:, :F2] + so[:, F2:]
    bi = bi_ref[...]
    counts = [jnp.sum(jnp.where(bi == b, 1.0, 0.0)) for b in range(B)]
    counts = jnp.stack(counts).reshape(B, 1)
    z = sums / jnp.maximum(counts, 1.0)
    h = jnp.dot(z, wh1_ref[...], preferred_element_type=jnp.float32)
    h = jnp.maximum(h + bh1_ref[...], 0.0)
    cn = (((1,), (1,)), ((), ()))
    out_ref[...] = (lax.dot_general(h, wh2t_ref[...], cn,
                                    preferred_element_type=jnp.float32)
                    + bh2_ref[...])


def _head(partials, bi_arr, Wh1, bh1, Wh2, bh2):
    return pl.pallas_call(
        _head_body,
        out_shape=jax.ShapeDtypeStruct((B, 2), jnp.float32),
    )(partials.reshape(NW * B // 2, 128), bi_arr,
      Wh1, bh1.reshape(1, 64), Wh2.T, bh2.reshape(1, 2))


def kernel(coords, feats, W1a, b1a, W1b, b1b, W2, b2, Wh1, bh1, Wh2, bh2):
    f2p, bip = _encoder(coords, feats, W1a, b1a, W1b, b1b, W2, b2)
    partials = _pool(bip.reshape(N), f2p)
    return _head(partials, bip, Wh1, bh1, Wh2, bh2)


# speedup vs baseline: 4.1935x; 1.0336x over previous
import functools

import jax
import jax.numpy as jnp
from jax import lax
from jax.experimental import pallas as pl
from jax.experimental.pallas import tpu as pltpu
from jax.experimental.pallas import tpu_sc as plsc

N = 32768
B = 16
F2 = 64
NC = 2
NS = 16
NW = NC * NS

ENC_BLK = 4096
GRID = N // ENC_BLK
ROWS = N // 2
CHUNK = 1024


def _encoder_body(coords_ref, feats_ref, w1a_ref, b1a_ref, w1b_ref, b1b_ref,
                  w2_ref, b2_ref, out_ref, bi_ref):
    x = feats_ref[...]
    cn = (((0,), (0,)), ((), ()))
    h = lax.dot_general(w1a_ref[...], x, cn, preferred_element_type=jnp.float32)
    h = jnp.maximum(h + jnp.transpose(b1a_ref[...]), 0.0)
    h = lax.dot_general(w1b_ref[...], h, cn, preferred_element_type=jnp.float32)
    h = jnp.maximum(h + jnp.transpose(b1b_ref[...]), 0.0)
    w2 = w2_ref[...]
    b2 = b2_ref[...]
    ha = lax.dot_general(h[:, :ENC_BLK // 2], w2, cn,
                         preferred_element_type=jnp.float32)
    hb = lax.dot_general(h[:, ENC_BLK // 2:], w2, cn,
                         preferred_element_type=jnp.float32)
    out_ref[:, :F2] = jnp.maximum(ha + b2, 0.0)
    out_ref[:, F2:] = jnp.maximum(hb + b2, 0.0)
    bi_ref[...] = coords_ref[...][0, :].reshape(ENC_BLK // 128, 128)


def _encoder(coords, feats, W1a, b1a, W1b, b1b, W2, b2):
    full = lambda shape: pl.BlockSpec(shape, lambda i: (0, 0))
    return pl.pallas_call(
        _encoder_body,
        grid=(GRID,),
        in_specs=[
            pl.BlockSpec((3, ENC_BLK), lambda i: (0, i)),
            pl.BlockSpec((8, ENC_BLK), lambda i: (0, i)),
            full((8, 16)), full((1, 16)),
            full((16, 32)), full((1, 32)),
            full((32, 64)), full((1, 64)),
        ],
        out_specs=(
            pl.BlockSpec((ENC_BLK // 2, 128), lambda i: (i, 0)),
            pl.BlockSpec((ENC_BLK // 128, 128), lambda i: (i, 0)),
        ),
        out_shape=(
            jax.ShapeDtypeStruct((ROWS, 128), jnp.float32),
            jax.ShapeDtypeStruct((N // 128, 128), jnp.int32),
        ),
    )(coords.T, feats.T, W1a, b1a.reshape(1, 16), W1b, b1b.reshape(1, 32),
      W2, b2.reshape(1, 64))


def _pool_body(bi_hbm, f2_hbm, out_hbm, idx_v, rows_v, zer_v, shared):
    c = lax.axis_index("c")
    s = lax.axis_index("s")
    wid = s * NC + c
    half = wid // 16
    t = wid % 16
    row0 = t * CHUNK
    p0 = (t // 2) * ENC_BLK + half * (ENC_BLK // 2) + (t % 2) * CHUNK

    pltpu.sync_copy(bi_hbm.at[pl.ds(p0, CHUNK)], idx_v)
    pltpu.sync_copy(f2_hbm.at[pl.ds(row0, CHUNK), pl.ds(half * F2, F2)],
                    rows_v)

    zero = jnp.zeros((16,), jnp.float32)
    for i in range(B):
        for j in range(F2 // 16):
            zer_v[i, pl.ds(j * 16, 16)] = zero
    pltpu.sync_copy(zer_v, shared.at[pl.ds(s * B, B), :])

    base = s * B
    for g in range(CHUNK // 16):
        idx_v[pl.ds(g * 16, 16)] = idx_v[pl.ds(g * 16, 16)] + base
    pltpu.sync_copy(rows_v, shared.at[idx_v], add=True)

    pltpu.sync_copy(shared.at[pl.ds(s * B, B), :],
                    out_hbm.at[pl.ds(wid * B, B), :])


def _pool(batch_idx_flat, f2_rows):
    mesh = plsc.VectorSubcoreMesh(core_axis_name="c", subcore_axis_name="s")
    f = functools.partial(
        pl.kernel,
        out_type=jax.ShapeDtypeStruct((NW * B, F2), jnp.float32),
        mesh=mesh,
        scratch_types=[
            pltpu.VMEM((CHUNK,), jnp.int32),
            pltpu.VMEM((CHUNK, F2), jnp.float32),
            pltpu.VMEM((B, F2), jnp.float32),
            pltpu.VMEM_SHARED((NS * B, F2), jnp.float32),
        ],
        compiler_params=pltpu.CompilerParams(use_tc_tiling_on_sc=False),
    )(_pool_body)
    return f(batch_idx_flat, f2_rows)


def _head_body(part_ref, bi_ref, wh1_ref, bh1_ref, wh2t_ref, bh2_ref, out_ref):
    x = part_ref[...]
    nr = NW * B // 2
    r = lax.broadcasted_iota(jnp.int32, (B, nr), 1)
    bcol = lax.broadcasted_iota(jnp.int32, (B, nr), 0)
    sel_e = ((2 * r) % B == bcol).astype(jnp.float32)
    sel_o = ((2 * r + 1) % B == bcol).astype(jnp.float32)
    se = jnp.dot(sel_e, x, preferred_element_type=jnp.float32)
    so = jnp.dot(sel_o, x, preferred_element_type=jnp.float32)
    sums = se[:, :F2] + so[:, F2:]
    bi = bi_ref[...]
    counts = [jnp.sum(jnp.where(bi == b, 1.0, 0.0)) for b in range(B)]
    counts = jnp.stack(counts).reshape(B, 1)
    z = sums / jnp.maximum(counts, 1.0)
    h = jnp.dot(z, wh1_ref[...], preferred_element_type=jnp.float32)
    h = jnp.maximum(h + bh1_ref[...], 0.0)
    cn = (((1,), (1,)), ((), ()))
    out_ref[...] = (lax.dot_general(h, wh2t_ref[...], cn,
                                    preferred_element_type=jnp.float32)
                    + bh2_ref[...])


def _head(partials, bi_arr, Wh1, bh1, Wh2, bh2):
    return pl.pallas_call(
        _head_body,
        out_shape=jax.ShapeDtypeStruct((B, 2), jnp.float32),
    )(partials.reshape(NW * B // 2, 128), bi_arr,
      Wh1, bh1.reshape(1, 64), Wh2.T, bh2.reshape(1, 2))


def kernel(coords, feats, W1a, b1a, W1b, b1b, W2, b2, Wh1, bh1, Wh2, bh2):
    f2p, bip = _encoder(coords, feats, W1a, b1a, W1b, b1b, W2, b2)
    partials = _pool(bip.reshape(N), f2p)
    return _head(partials, bip, Wh1, bh1, Wh2, bh2)
